# Initial kernel scaffold; baseline (speedup 1.0000x reference)
import jax
import jax.numpy as jnp
from jax.experimental import pallas as pl


def kernel(ent_embs, rel_table, rels, neg_idx):
    raise NotImplementedError("write your pallas kernel here")


# speedup vs baseline: 8.1200x; 8.1200x over previous
import functools

import jax
import jax.numpy as jnp
from jax import lax
from jax.experimental import pallas as pl
from jax.experimental.pallas import tpu as pltpu
from jax.experimental.pallas import tpu_sc as plsc

DIM = 128
HALF = DIM // 2
B = 4096
NUM_NEG = 64
RPB = 2 * NUM_NEG
NW = 32
BS = B // NW
LANES = 16
PCOLS = NUM_NEG * LANES
REGULARIZER = 0.01


def _sc_body(ent_hbm, negidx_hbm, relidx_hbm, reltab_hbm,
             part_hbm, relvec_hbm,
             idx_v, relidx_v, rel_v, rows0_v, rows1_v, part0_v, part1_v,
             sem_rel, sem_r0, sem_r1, sem_p0, sem_p1):
    c = lax.axis_index("c")
    s = lax.axis_index("s")
    wid = s * 2 + c
    base = wid * BS

    pltpu.sync_copy(negidx_hbm.at[pl.ds(base, BS)], idx_v.at[pl.ds(0, BS)])
    for k in range(RPB // 16):
        idx_v[BS, pl.ds(k * 16, 16)] = jnp.zeros((16,), jnp.int32)
    pltpu.sync_copy(relidx_hbm.at[pl.ds(base, BS)], relidx_v)
    pltpu.async_copy(reltab_hbm.at[relidx_v], rel_v, sem_rel).wait()
    pltpu.sync_copy(rel_v, relvec_hbm.at[pl.ds(base, BS)])

    rows = (rows0_v, rows1_v)
    rsem = (sem_r0, sem_r1)
    part = (part0_v, part1_v)
    psem = (sem_p0, sem_p1)

    pltpu.async_copy(ent_hbm.at[idx_v.at[0]], rows0_v, sem_r0)

    def g_body(g, carry):
        for p in (0, 1):
            b = 2 * g + p
            pltpu.async_copy(ent_hbm.at[idx_v.at[b + 1]], rows[1 - p],
                             rsem[1 - p])
            pltpu.make_async_copy(ent_hbm.at[idx_v.at[b]], rows[p],
                                  rsem[p]).wait()

            @pl.when(g > 0)
            def _():
                pltpu.make_async_copy(part[p], part_hbm.at[base + b - 2],
                                      psem[p]).wait()

            rows_v = rows[p]
            part_v = part[p]
            rel = [rel_v[b, pl.ds(k * 16, 16)] for k in range(8)]

            @plsc.parallel_loop(0, NUM_NEG, unroll=2)
            def j_body(j):
                terms = []
                for k in range(4):
                    hr = rows_v[2 * j, pl.ds(k * 16, 16)]
                    hi = rows_v[2 * j, pl.ds((k + 4) * 16, 16)]
                    tr = rows_v[2 * j + 1, pl.ds(k * 16, 16)]
                    ti = rows_v[2 * j + 1, pl.ds((k + 4) * 16, 16)]
                    a = hr * tr + hi * ti
                    cc = hr * ti - hi * tr
                    terms.append(rel[k] * a + rel[k + 4] * cc)
                part_v[pl.ds(j * 16, 16)] = ((terms[0] + terms[1])
                                             + (terms[2] + terms[3]))

            pltpu.async_copy(part_v, part_hbm.at[base + b], psem[p])
        return carry

    lax.fori_loop(0, BS // 2, g_body, 0)
    pltpu.make_async_copy(part0_v, part_hbm.at[base + BS - 2], sem_p0).wait()
    pltpu.make_async_copy(part1_v, part_hbm.at[base + BS - 1], sem_p1).wait()
    pltpu.make_async_copy(ent_hbm.at[idx_v.at[BS]], rows0_v, sem_r0).wait()


_sc_call = functools.partial(
    pl.kernel,
    mesh=plsc.VectorSubcoreMesh(core_axis_name="c", subcore_axis_name="s"),
    out_type=[
        jax.ShapeDtypeStruct((B, PCOLS), jnp.float32),
        jax.ShapeDtypeStruct((B, DIM), jnp.float32),
    ],
    scratch_types=[
        pltpu.VMEM((BS + 1, RPB), jnp.int32),
        pltpu.VMEM((BS,), jnp.int32),
        pltpu.VMEM((BS, DIM), jnp.float32),
        pltpu.VMEM((RPB, DIM), jnp.float32),
        pltpu.VMEM((RPB, DIM), jnp.float32),
        pltpu.VMEM((PCOLS,), jnp.float32),
        pltpu.VMEM((PCOLS,), jnp.float32),
        pltpu.SemaphoreType.DMA,
        pltpu.SemaphoreType.DMA,
        pltpu.SemaphoreType.DMA,
        pltpu.SemaphoreType.DMA,
        pltpu.SemaphoreType.DMA,
    ],
)(_sc_body)


def _tc_body(heads_ref, tails_ref, relv_ref, part_ref, out_ref):
    h = heads_ref[...]
    t = tails_ref[...]
    r = relv_ref[...]
    hr, hi = h[:, :HALF], h[:, HALF:]
    tr, ti = t[:, :HALF], t[:, HALF:]
    rr, ri = r[:, :HALF], r[:, HALF:]
    pos = jnp.sum(rr * hr * tr + rr * hi * ti + ri * hr * ti - ri * hi * tr,
                  axis=-1)
    reg = (jnp.mean(h * h) + jnp.mean(t * t) + jnp.mean(r * r)) / 3.0

    row = lax.broadcasted_iota(jnp.int32, (PCOLS, NUM_NEG), 0)
    col = lax.broadcasted_iota(jnp.int32, (PCOLS, NUM_NEG), 1)
    fold = jnp.where(row // LANES == col, 1.0, 0.0).astype(jnp.float32)
    ns = jnp.dot(part_ref[...], fold, preferred_element_type=jnp.float32)

    def softplus(x):
        return jnp.maximum(x, 0.0) + jnp.log1p(jnp.exp(-jnp.abs(x)))

    model = (jnp.mean(softplus(-pos)) + jnp.mean(softplus(ns))) / 2.0
    out_ref[...] = jnp.full((1, 1), model + REGULARIZER * reg, jnp.float32)


def kernel(ent_embs, rel_table, rels, neg_idx):
    ent_flat = ent_embs.reshape(2 * B, DIM)
    negidx = neg_idx.astype(jnp.int32).reshape(B, RPB)
    relidx = rels.astype(jnp.int32).reshape(B)
    partials, rel_vecs = _sc_call(ent_flat, negidx, relidx, rel_table)
    heads = ent_embs[:, 0, :]
    tails = ent_embs[:, 1, :]
    out = pl.pallas_call(
        _tc_body,
        out_shape=jax.ShapeDtypeStruct((1, 1), jnp.float32),
    )(heads, tails, rel_vecs, partials)
    return out[0, 0]
